# Initial kernel scaffold; baseline (speedup 1.0000x reference)
import jax
import jax.numpy as jnp
from jax.experimental import pallas as pl


def kernel(x, edge_values, W, edge_index):
    raise NotImplementedError("write your pallas kernel here")


# speedup vs baseline: 2.4940x; 2.4940x over previous
import functools

import jax
import jax.numpy as jnp
from jax import lax
from jax.experimental import pallas as pl
from jax.experimental.pallas import tpu as pltpu
from jax.experimental.pallas import tpu_sc as plsc

N_NODES = 10000
N_EDGES = 320000
D = 128
DH = D // 2

NC = 2
NS = 16
EPT = N_EDGES // NS
C = 80
NCHUNK = EPT // C
RPT = 640
RPT_LAST = N_NODES - (NS - 1) * RPT


def _edge_pipeline(x_hbm, row_hbm, col_hbm, val_hbm,
                   colbuf, rowbuf, valbuf, rowsbuf, acc, sem, sid):
    ebase = sid * EPT

    def chunk(ci, carry):
        base = ebase + ci * C
        pltpu.sync_copy(col_hbm.at[pl.ds(base, C)], colbuf)
        pltpu.sync_copy(row_hbm.at[pl.ds(base, C)], rowbuf)
        pltpu.sync_copy(val_hbm.at[pl.ds(base, C)], valbuf)
        pltpu.async_copy(x_hbm.at[colbuf], rowsbuf, sem).wait()

        def scale(i, c2):
            v = plsc.load_gather(valbuf, [jnp.full((16,), i, jnp.int32)])
            for j in range(DH // 16):
                sl = pl.ds(j * 16, 16)
                rowsbuf[i, sl] = rowsbuf[i, sl] * v
            return c2

        lax.fori_loop(0, C, scale, 0)
        pltpu.sync_copy(rowsbuf, acc.at[rowbuf], add=True)
        return carry

    lax.fori_loop(0, NCHUNK, chunk, 0)


def _sc_body(xlo_hbm, xhi_hbm, row_hbm, col_hbm, val_hbm, out_hbm,
             colbuf, rowbuf, valbuf, rowsbuf, outbuf, acc, sem):
    cid = lax.axis_index("c")
    sid = lax.axis_index("s")

    zero16 = jnp.zeros((16,), jnp.float32)

    def zrow(i, carry):
        for j in range(DH // 16):
            outbuf[i, pl.ds(j * 16, 16)] = zero16
        return carry

    lax.fori_loop(0, RPT, zrow, 0)

    @pl.when(sid < NS - 1)
    def _():
        pltpu.sync_copy(outbuf, acc.at[pl.ds(sid * RPT, RPT)])

    @pl.when(sid == NS - 1)
    def _():
        pltpu.sync_copy(outbuf.at[pl.ds(0, RPT_LAST)],
                        acc.at[pl.ds((NS - 1) * RPT, RPT_LAST)])

    plsc.subcore_barrier()

    @pl.when(cid == 0)
    def _():
        _edge_pipeline(xlo_hbm, row_hbm, col_hbm, val_hbm,
                       colbuf, rowbuf, valbuf, rowsbuf, acc, sem, sid)

    @pl.when(cid == 1)
    def _():
        _edge_pipeline(xhi_hbm, row_hbm, col_hbm, val_hbm,
                       colbuf, rowbuf, valbuf, rowsbuf, acc, sem, sid)

    plsc.subcore_barrier()

    @pl.when(sid < NS - 1)
    def _():
        pltpu.sync_copy(acc.at[pl.ds(sid * RPT, RPT)], outbuf)
        pltpu.sync_copy(outbuf, out_hbm.at[cid, pl.ds(sid * RPT, RPT)])

    @pl.when(sid == NS - 1)
    def _():
        base = (NS - 1) * RPT
        pltpu.sync_copy(acc.at[pl.ds(base, RPT_LAST)],
                        outbuf.at[pl.ds(0, RPT_LAST)])
        pltpu.sync_copy(outbuf.at[pl.ds(0, RPT_LAST)],
                        out_hbm.at[cid, pl.ds(base, RPT_LAST)])


_sc_aggregate = functools.partial(
    pl.kernel,
    out_type=jax.ShapeDtypeStruct((NC, N_NODES, DH), jnp.float32),
    mesh=plsc.VectorSubcoreMesh(core_axis_name="c", subcore_axis_name="s"),
    scratch_types=[
        pltpu.VMEM((C,), jnp.int32),
        pltpu.VMEM((C,), jnp.int32),
        pltpu.VMEM((C,), jnp.float32),
        pltpu.VMEM((C, DH), jnp.float32),
        pltpu.VMEM((RPT, DH), jnp.float32),
        pltpu.VMEM_SHARED((N_NODES, DH), jnp.float32),
        pltpu.SemaphoreType.DMA,
    ],
    compiler_params=pltpu.CompilerParams(needs_layout_passes=False,
                                         use_tc_tiling_on_sc=False),
)(_sc_body)


def _tc_body(p_ref, w_ref, o_ref):
    acc = (lax.dot(p_ref[0], w_ref[pl.ds(0, DH), :],
                   precision=lax.Precision.HIGHEST,
                   preferred_element_type=jnp.float32)
           + lax.dot(p_ref[1], w_ref[pl.ds(DH, DH), :],
                     precision=lax.Precision.HIGHEST,
                     preferred_element_type=jnp.float32))
    o_ref[...] = jnp.maximum(acc, 0.0)


def _tc_finalize(agg, W):
    G = 10
    BM = N_NODES // G
    return pl.pallas_call(
        _tc_body,
        grid=(G,),
        in_specs=[
            pl.BlockSpec((NC, BM, DH), lambda i: (0, i, 0)),
            pl.BlockSpec((D, D), lambda i: (0, 0)),
        ],
        out_specs=pl.BlockSpec((BM, D), lambda i: (i, 0)),
        out_shape=jax.ShapeDtypeStruct((N_NODES, D), jnp.float32),
    )(agg, W)


def kernel(x, edge_values, W, edge_index):
    ei = edge_index.astype(jnp.int32)
    row = ei[0]
    col = ei[1]
    agg = _sc_aggregate(x[:, :DH], x[:, DH:], row, col, edge_values)
    return _tc_finalize(agg, W)


# speedup vs baseline: 8.1822x; 3.2807x over previous
import functools

import jax
import jax.numpy as jnp
from jax import lax
from jax.experimental import pallas as pl
from jax.experimental.pallas import tpu as pltpu
from jax.experimental.pallas import tpu_sc as plsc

N_NODES = 10000
N_EDGES = 320000
D = 128
DH = D // 2

NC = 2
NS = 16
EPT = N_EDGES // NS
C = 80
NCHUNK = EPT // C
RPT = 640
RPT_LAST = N_NODES - (NS - 1) * RPT


def _edge_pipeline(x_hbm, colbuf, rowbuf, valbuf,
                   rowsA, rowsB, acc, gsemA, gsemB, ssemA, ssemB):
    def gather_start(ci, rows, sem):
        pltpu.async_copy(x_hbm.at[colbuf.at[ci]], rows, sem)

    def gather_wait(ci, rows, sem):
        pltpu.make_async_copy(x_hbm.at[colbuf.at[ci]], rows, sem).wait()

    def scat_start(ci, rows, sem):
        pltpu.async_copy(rows, acc.at[rowbuf.at[ci]], sem, add=True)

    def scat_wait(ci, rows, sem):
        pltpu.make_async_copy(rows, acc.at[rowbuf.at[ci]], sem).wait()

    def scale(rows, ci):
        @plsc.parallel_loop(0, C, 1, unroll=4)
        def _(i):
            v = plsc.load_gather(
                valbuf, [jnp.full((16,), ci, jnp.int32),
                         jnp.full((16,), i, jnp.int32)])
            for j in range(DH // 16):
                sl = pl.ds(j * 16, 16)
                rows[i, sl] = rows[i, sl] * v

    gather_start(0, rowsA, gsemA)

    def step(k, carry):
        ci0 = 2 * k
        ci1 = 2 * k + 1

        @pl.when(k > 0)
        def _():
            scat_wait(ci1 - 2, rowsB, ssemB)

        gather_start(ci1, rowsB, gsemB)
        gather_wait(ci0, rowsA, gsemA)
        scale(rowsA, ci0)
        scat_start(ci0, rowsA, ssemA)

        @pl.when(k < NCHUNK // 2 - 1)
        def _():
            scat_wait(ci0, rowsA, ssemA)
            gather_start(ci0 + 2, rowsA, gsemA)

        gather_wait(ci1, rowsB, gsemB)
        scale(rowsB, ci1)
        scat_start(ci1, rowsB, ssemB)
        return carry

    lax.fori_loop(0, NCHUNK // 2, step, 0)
    scat_wait(NCHUNK - 2, rowsA, ssemA)
    scat_wait(NCHUNK - 1, rowsB, ssemB)


def _sc_body(xlo_hbm, xhi_hbm, row_hbm, col_hbm, val_hbm, out_hbm,
             colbuf, rowbuf, valbuf, rowsA, rowsB, acc,
             gsemA, gsemB, ssemA, ssemB):
    cid = lax.axis_index("c")
    sid = lax.axis_index("s")

    zero16 = jnp.zeros((16,), jnp.float32)

    def zrow(i, carry):
        for j in range(DH // 16):
            rowsA[i, pl.ds(j * 16, 16)] = zero16
        return carry

    lax.fori_loop(0, C, zrow, 0)

    @pl.when(sid < NS - 1)
    def _():
        for s in range(RPT // C):
            pltpu.sync_copy(rowsA, acc.at[pl.ds(sid * RPT + s * C, C)])

    @pl.when(sid == NS - 1)
    def _():
        for s in range(RPT_LAST // C):
            pltpu.sync_copy(rowsA, acc.at[pl.ds((NS - 1) * RPT + s * C, C)])

    pltpu.sync_copy(row_hbm.at[sid], rowbuf)
    pltpu.sync_copy(col_hbm.at[sid], colbuf)
    pltpu.sync_copy(val_hbm.at[sid], valbuf)

    plsc.subcore_barrier()

    @pl.when(cid == 0)
    def _():
        _edge_pipeline(xlo_hbm, colbuf, rowbuf, valbuf,
                       rowsA, rowsB, acc, gsemA, gsemB, ssemA, ssemB)

    @pl.when(cid == 1)
    def _():
        _edge_pipeline(xhi_hbm, colbuf, rowbuf, valbuf,
                       rowsA, rowsB, acc, gsemA, gsemB, ssemA, ssemB)

    plsc.subcore_barrier()

    def dump(nslices):
        for s in range(nslices):
            buf = rowsA if s % 2 == 0 else rowsB
            base = sid * RPT + s * C
            pltpu.sync_copy(acc.at[pl.ds(base, C)], buf)
            pltpu.sync_copy(buf, out_hbm.at[cid, pl.ds(base, C)])

    @pl.when(sid < NS - 1)
    def _():
        dump(RPT // C)

    @pl.when(sid == NS - 1)
    def _():
        dump(RPT_LAST // C)


_sc_aggregate = functools.partial(
    pl.kernel,
    out_type=jax.ShapeDtypeStruct((NC, N_NODES, DH), jnp.float32),
    mesh=plsc.VectorSubcoreMesh(core_axis_name="c", subcore_axis_name="s"),
    scratch_types=[
        pltpu.VMEM((NCHUNK, C), jnp.int32),
        pltpu.VMEM((NCHUNK, C), jnp.int32),
        pltpu.VMEM((NCHUNK, C), jnp.float32),
        pltpu.VMEM((C, DH), jnp.float32),
        pltpu.VMEM((C, DH), jnp.float32),
        pltpu.VMEM_SHARED((N_NODES, DH), jnp.float32),
        pltpu.SemaphoreType.DMA,
        pltpu.SemaphoreType.DMA,
        pltpu.SemaphoreType.DMA,
        pltpu.SemaphoreType.DMA,
    ],
    compiler_params=pltpu.CompilerParams(needs_layout_passes=False,
                                         use_tc_tiling_on_sc=False),
)(_sc_body)


def _tc_body(p_ref, w_ref, o_ref):
    acc = (lax.dot(p_ref[0], w_ref[pl.ds(0, DH), :],
                   precision=lax.Precision.HIGHEST,
                   preferred_element_type=jnp.float32)
           + lax.dot(p_ref[1], w_ref[pl.ds(DH, DH), :],
                     precision=lax.Precision.HIGHEST,
                     preferred_element_type=jnp.float32))
    o_ref[...] = jnp.maximum(acc, 0.0)


def _tc_finalize(agg, W):
    G = 10
    BM = N_NODES // G
    return pl.pallas_call(
        _tc_body,
        grid=(G,),
        in_specs=[
            pl.BlockSpec((NC, BM, DH), lambda i: (0, i, 0)),
            pl.BlockSpec((D, D), lambda i: (0, 0)),
        ],
        out_specs=pl.BlockSpec((BM, D), lambda i: (i, 0)),
        out_shape=jax.ShapeDtypeStruct((N_NODES, D), jnp.float32),
    )(agg, W)


def kernel(x, edge_values, W, edge_index):
    ei = edge_index.astype(jnp.int32)
    row = ei[0].reshape(NS, NCHUNK, C)
    col = ei[1].reshape(NS, NCHUNK, C)
    val = edge_values.reshape(NS, NCHUNK, C)
    agg = _sc_aggregate(x[:, :DH], x[:, DH:], row, col, val)
    return _tc_finalize(agg, W)


# speedup vs baseline: 10.9094x; 1.3333x over previous
import functools

import jax
import jax.numpy as jnp
from jax import lax
from jax.experimental import pallas as pl
from jax.experimental.pallas import tpu as pltpu
from jax.experimental.pallas import tpu_sc as plsc

N_NODES = 10000
N_EDGES = 320000
D = 128
DH = D // 2

NC = 2
NS = 16
EPT = N_EDGES // NS
C = 80
NCHUNK = EPT // C
RPT = 640
RPT_LAST = N_NODES - (NS - 1) * RPT


NBUF = 5


def _edge_pipeline(x_hbm, colbuf, rowbuf, valbuf, bufs, acc, gsems, ssems):
    def gather_start(ci, rows, sem):
        pltpu.async_copy(x_hbm.at[colbuf.at[ci]], rows, sem)

    def gather_wait(ci, rows, sem):
        pltpu.make_async_copy(x_hbm.at[colbuf.at[ci]], rows, sem).wait()

    def scat_start(ci, rows, sem):
        pltpu.async_copy(rows, acc.at[rowbuf.at[ci]], sem, add=True)

    def scat_wait(ci, rows, sem):
        pltpu.make_async_copy(rows, acc.at[rowbuf.at[ci]], sem).wait()

    def scale(rows, ci):
        @plsc.parallel_loop(0, C, 1, unroll=4)
        def _(i):
            v = plsc.load_gather(
                valbuf, [jnp.full((16,), ci, jnp.int32),
                         jnp.full((16,), i, jnp.int32)])
            for j in range(DH // 16):
                sl = pl.ds(j * 16, 16)
                rows[i, sl] = rows[i, sl] * v

    for b in range(3):
        gather_start(b, bufs[b], gsems[b])

    def step(k, carry):
        for j in range(NBUF):
            ci = NBUF * k + j
            jj = (j + 3) % NBUF

            if j >= 2:
                scat_wait(ci - 2, bufs[jj], ssems[jj])
            else:
                @pl.when(k > 0)
                def _():
                    scat_wait(ci - 2, bufs[jj], ssems[jj])

            if j < 2:
                gather_start(ci + 3, bufs[jj], gsems[jj])
            else:
                @pl.when(ci + 3 < NCHUNK)
                def _():
                    gather_start(ci + 3, bufs[jj], gsems[jj])

            gather_wait(ci, bufs[j], gsems[j])
            scale(bufs[j], ci)
            scat_start(ci, bufs[j], ssems[j])
        return carry

    lax.fori_loop(0, NCHUNK // NBUF, step, 0)
    scat_wait(NCHUNK - 2, bufs[3], ssems[3])
    scat_wait(NCHUNK - 1, bufs[4], ssems[4])


def _sc_body(xlo_hbm, xhi_hbm, row_hbm, col_hbm, val_hbm, out_hbm,
             colbuf, rowbuf, valbuf,
             rows0, rows1, rows2, rows3, rows4, acc,
             gsem0, gsem1, gsem2, gsem3, gsem4,
             ssem0, ssem1, ssem2, ssem3, ssem4):
    bufs = (rows0, rows1, rows2, rows3, rows4)
    gsems = (gsem0, gsem1, gsem2, gsem3, gsem4)
    ssems = (ssem0, ssem1, ssem2, ssem3, ssem4)
    rowsA, rowsB = rows0, rows1
    cid = lax.axis_index("c")
    sid = lax.axis_index("s")

    zero16 = jnp.zeros((16,), jnp.float32)

    def zrow(i, carry):
        for j in range(DH // 16):
            rowsA[i, pl.ds(j * 16, 16)] = zero16
        return carry

    lax.fori_loop(0, C, zrow, 0)

    @pl.when(sid < NS - 1)
    def _():
        for s in range(RPT // C):
            pltpu.sync_copy(rowsA, acc.at[pl.ds(sid * RPT + s * C, C)])

    @pl.when(sid == NS - 1)
    def _():
        for s in range(RPT_LAST // C):
            pltpu.sync_copy(rowsA, acc.at[pl.ds((NS - 1) * RPT + s * C, C)])

    pltpu.sync_copy(row_hbm.at[sid], rowbuf)
    pltpu.sync_copy(col_hbm.at[sid], colbuf)
    pltpu.sync_copy(val_hbm.at[sid], valbuf)

    plsc.subcore_barrier()

    @pl.when(cid == 0)
    def _():
        _edge_pipeline(xlo_hbm, colbuf, rowbuf, valbuf, bufs, acc,
                       gsems, ssems)

    @pl.when(cid == 1)
    def _():
        _edge_pipeline(xhi_hbm, colbuf, rowbuf, valbuf, bufs, acc,
                       gsems, ssems)

    plsc.subcore_barrier()

    def dump(nslices):
        for s in range(nslices):
            buf = rowsA if s % 2 == 0 else rowsB
            base = sid * RPT + s * C
            pltpu.sync_copy(acc.at[pl.ds(base, C)], buf)
            pltpu.sync_copy(buf, out_hbm.at[cid, pl.ds(base, C)])

    @pl.when(sid < NS - 1)
    def _():
        dump(RPT // C)

    @pl.when(sid == NS - 1)
    def _():
        dump(RPT_LAST // C)


_sc_aggregate = functools.partial(
    pl.kernel,
    out_type=jax.ShapeDtypeStruct((NC, N_NODES, DH), jnp.float32),
    mesh=plsc.VectorSubcoreMesh(core_axis_name="c", subcore_axis_name="s"),
    scratch_types=[
        pltpu.VMEM((NCHUNK, C), jnp.int32),
        pltpu.VMEM((NCHUNK, C), jnp.int32),
        pltpu.VMEM((NCHUNK, C), jnp.float32),
        pltpu.VMEM((C, DH), jnp.float32),
        pltpu.VMEM((C, DH), jnp.float32),
        pltpu.VMEM((C, DH), jnp.float32),
        pltpu.VMEM((C, DH), jnp.float32),
        pltpu.VMEM((C, DH), jnp.float32),
        pltpu.VMEM_SHARED((N_NODES, DH), jnp.float32),
    ] + [pltpu.SemaphoreType.DMA] * 10,
    compiler_params=pltpu.CompilerParams(needs_layout_passes=False,
                                         use_tc_tiling_on_sc=False),
)(_sc_body)


def _tc_body(p_ref, w_ref, o_ref):
    acc = (lax.dot(p_ref[0], w_ref[pl.ds(0, DH), :],
                   precision=lax.Precision.HIGHEST,
                   preferred_element_type=jnp.float32)
           + lax.dot(p_ref[1], w_ref[pl.ds(DH, DH), :],
                     precision=lax.Precision.HIGHEST,
                     preferred_element_type=jnp.float32))
    o_ref[...] = jnp.maximum(acc, 0.0)


def _tc_finalize(agg, W):
    G = 10
    BM = N_NODES // G
    return pl.pallas_call(
        _tc_body,
        grid=(G,),
        in_specs=[
            pl.BlockSpec((NC, BM, DH), lambda i: (0, i, 0)),
            pl.BlockSpec((D, D), lambda i: (0, 0)),
        ],
        out_specs=pl.BlockSpec((BM, D), lambda i: (i, 0)),
        out_shape=jax.ShapeDtypeStruct((N_NODES, D), jnp.float32),
    )(agg, W)


def kernel(x, edge_values, W, edge_index):
    ei = edge_index.astype(jnp.int32)
    row = ei[0].reshape(NS, NCHUNK, C)
    col = ei[1].reshape(NS, NCHUNK, C)
    val = edge_values.reshape(NS, NCHUNK, C)
    agg = _sc_aggregate(x[:, :DH], x[:, DH:], row, col, val)
    return _tc_finalize(agg, W)


# speedup vs baseline: 12.2943x; 1.1269x over previous
import functools

import jax
import jax.numpy as jnp
from jax import lax
from jax.experimental import pallas as pl
from jax.experimental.pallas import tpu as pltpu
from jax.experimental.pallas import tpu_sc as plsc

N_NODES = 10000
N_EDGES = 320000
D = 128
DH = D // 2

NC = 2
NS = 16
EPT = N_EDGES // NS
C = 80
NCHUNK = EPT // C
RPT = 640
RPT_LAST = N_NODES - (NS - 1) * RPT


NBUF = 5


def _edge_pipeline(x_hbm, colbuf, rowbuf, valbuf, bufs, acc, gsems, ssems):
    def gather_start(ci, rows, sem):
        pltpu.async_copy(x_hbm.at[colbuf.at[ci]], rows, sem)

    def gather_wait(ci, rows, sem):
        pltpu.make_async_copy(x_hbm.at[colbuf.at[ci]], rows, sem).wait()

    def scat_start(ci, rows, sem):
        pltpu.async_copy(rows, acc.at[rowbuf.at[ci]], sem, add=True)

    def scat_wait(ci, rows, sem):
        pltpu.make_async_copy(rows, acc.at[rowbuf.at[ci]], sem).wait()

    def scale(rows, ci):
        @plsc.parallel_loop(0, C, 1, unroll=8)
        def _(i):
            v = plsc.load_gather(
                valbuf, [jnp.full((16,), ci, jnp.int32),
                         jnp.full((16,), i, jnp.int32)])
            for j in range(DH // 16):
                sl = pl.ds(j * 16, 16)
                rows[i, sl] = rows[i, sl] * v

    for b in range(3):
        gather_start(b, bufs[b], gsems[b])

    def step(k, carry):
        for j in range(NBUF):
            ci = NBUF * k + j
            jj = (j + 3) % NBUF

            if j >= 2:
                scat_wait(ci - 2, bufs[jj], ssems[jj])
            else:
                @pl.when(k > 0)
                def _():
                    scat_wait(ci - 2, bufs[jj], ssems[jj])

            if j < 2:
                gather_start(ci + 3, bufs[jj], gsems[jj])
            else:
                @pl.when(ci + 3 < NCHUNK)
                def _():
                    gather_start(ci + 3, bufs[jj], gsems[jj])

            gather_wait(ci, bufs[j], gsems[j])
            scale(bufs[j], ci)
            scat_start(ci, bufs[j], ssems[j])
        return carry

    lax.fori_loop(0, NCHUNK // NBUF, step, 0)
    scat_wait(NCHUNK - 2, bufs[3], ssems[3])
    scat_wait(NCHUNK - 1, bufs[4], ssems[4])


def _sc_body(x2_hbm, row_hbm, col_hbm, val_hbm, out_hbm,
             colbuf, rowbuf, valbuf,
             rows0, rows1, rows2, rows3, rows4, acc,
             gsem0, gsem1, gsem2, gsem3, gsem4,
             ssem0, ssem1, ssem2, ssem3, ssem4):
    bufs = (rows0, rows1, rows2, rows3, rows4)
    gsems = (gsem0, gsem1, gsem2, gsem3, gsem4)
    ssems = (ssem0, ssem1, ssem2, ssem3, ssem4)
    rowsA, rowsB = rows0, rows1
    cid = lax.axis_index("c")
    sid = lax.axis_index("s")

    zero16 = jnp.zeros((16,), jnp.float32)

    def zrow(i, carry):
        for j in range(DH // 16):
            rowsA[i, pl.ds(j * 16, 16)] = zero16
        return carry

    lax.fori_loop(0, C, zrow, 0)

    @pl.when(sid < NS - 1)
    def _():
        for s in range(RPT // C):
            pltpu.sync_copy(rowsA, acc.at[pl.ds(sid * RPT + s * C, C)])

    @pl.when(sid == NS - 1)
    def _():
        for s in range(RPT_LAST // C):
            pltpu.sync_copy(rowsA, acc.at[pl.ds((NS - 1) * RPT + s * C, C)])

    pltpu.sync_copy(row_hbm.at[sid], rowbuf)
    pltpu.sync_copy(col_hbm.at[sid], colbuf)
    pltpu.sync_copy(val_hbm.at[sid], valbuf)

    def xform(i, carry):
        for g in range(C // 16):
            sl = pl.ds(g * 16, 16)
            v = colbuf[i, sl]
            colbuf[i, sl] = v + v + cid
        return carry

    lax.fori_loop(0, NCHUNK, xform, 0)

    plsc.subcore_barrier()

    _edge_pipeline(x2_hbm, colbuf, rowbuf, valbuf, bufs, acc, gsems, ssems)

    plsc.subcore_barrier()

    def dump(nslices):
        for s in range(nslices):
            buf = rowsA if s % 2 == 0 else rowsB
            base = sid * RPT + s * C
            pltpu.sync_copy(acc.at[pl.ds(base, C)], buf)
            pltpu.sync_copy(buf, out_hbm.at[cid, pl.ds(base, C)])

    @pl.when(sid < NS - 1)
    def _():
        dump(RPT // C)

    @pl.when(sid == NS - 1)
    def _():
        dump(RPT_LAST // C)


_sc_aggregate = functools.partial(
    pl.kernel,
    out_type=jax.ShapeDtypeStruct((NC, N_NODES, DH), jnp.float32),
    mesh=plsc.VectorSubcoreMesh(core_axis_name="c", subcore_axis_name="s"),
    scratch_types=[
        pltpu.VMEM((NCHUNK, C), jnp.int32),
        pltpu.VMEM((NCHUNK, C), jnp.int32),
        pltpu.VMEM((NCHUNK, C), jnp.float32),
        pltpu.VMEM((C, DH), jnp.float32),
        pltpu.VMEM((C, DH), jnp.float32),
        pltpu.VMEM((C, DH), jnp.float32),
        pltpu.VMEM((C, DH), jnp.float32),
        pltpu.VMEM((C, DH), jnp.float32),
        pltpu.VMEM_SHARED((N_NODES, DH), jnp.float32),
    ] + [pltpu.SemaphoreType.DMA] * 10,
    compiler_params=pltpu.CompilerParams(needs_layout_passes=False,
                                         use_tc_tiling_on_sc=False),
)(_sc_body)


def _tc_body(p_ref, w_ref, o_ref):
    acc = (lax.dot(p_ref[0], w_ref[pl.ds(0, DH), :],
                   precision=lax.Precision.DEFAULT,
                   preferred_element_type=jnp.float32)
           + lax.dot(p_ref[1], w_ref[pl.ds(DH, DH), :],
                     precision=lax.Precision.DEFAULT,
                     preferred_element_type=jnp.float32))
    o_ref[...] = jnp.maximum(acc, 0.0)


def _tc_finalize(agg, W):
    G = 10
    BM = N_NODES // G
    return pl.pallas_call(
        _tc_body,
        grid=(G,),
        in_specs=[
            pl.BlockSpec((NC, BM, DH), lambda i: (0, i, 0)),
            pl.BlockSpec((D, D), lambda i: (0, 0)),
        ],
        out_specs=pl.BlockSpec((BM, D), lambda i: (i, 0)),
        out_shape=jax.ShapeDtypeStruct((N_NODES, D), jnp.float32),
    )(agg, W)


def kernel(x, edge_values, W, edge_index):
    ei = edge_index.astype(jnp.int32)
    row = ei[0].reshape(NS, NCHUNK, C)
    col = ei[1].reshape(NS, NCHUNK, C)
    val = edge_values.reshape(NS, NCHUNK, C)
    agg = _sc_aggregate(x.reshape(2 * N_NODES, DH), row, col, val)
    return _tc_finalize(agg, W)
